# Initial kernel scaffold; baseline (speedup 1.0000x reference)
import jax
import jax.numpy as jnp
from jax.experimental import pallas as pl


def kernel(node_feat, edge_candidate, num_edge_candidate, nnodes, W_enc, b_enc, W1, b1, W2, b2):
    raise NotImplementedError("write your pallas kernel here")


# speedup vs baseline: 5.0040x; 5.0040x over previous
import functools

import jax
import jax.numpy as jnp
from jax import lax
from jax.experimental import pallas as pl
from jax.experimental.pallas import tpu as pltpu
from jax.experimental.pallas import tpu_sc as plsc

N_NODES = 10000
E_TOTAL = 320000
D = 128
HID = 64

NC = 2
NS = 16
NW = NC * NS
E_PER_W = E_TOTAL // NW
CHUNK = 400
N_CHUNKS = E_PER_W // CHUNK


def _node_projections(node_feat, W_enc, b_enc, W1, b1):
    def body(nf, we, be, w1c, bc, t_out):
        x = jnp.dot(nf[:], we[:], preferred_element_type=jnp.float32) + be[:]
        t_out[:] = jnp.dot(x, w1c[:], preferred_element_type=jnp.float32) + bc[:]

    w1cat = jnp.concatenate([W1[:D, :], W1[D:, :]], axis=1)
    bcat = jnp.concatenate([b1, jnp.zeros((HID,), jnp.float32)])
    return pl.pallas_call(
        body,
        out_shape=jax.ShapeDtypeStruct((N_NODES, 2 * HID), jnp.float32),
    )(node_feat, W_enc, b_enc.reshape(1, D), w1cat, bcat.reshape(1, 2 * HID))


def _edge_scores(T, src, dst, w2):
    mesh = plsc.VectorSubcoreMesh(
        core_axis_name="c", subcore_axis_name="s", num_cores=NC, num_subcores=NS
    )

    @functools.partial(
        pl.kernel,
        out_type=jax.ShapeDtypeStruct((E_TOTAL,), jnp.float32),
        mesh=mesh,
        compiler_params=pltpu.CompilerParams(needs_layout_passes=False),
        scratch_types=[
            pltpu.VMEM((CHUNK,), jnp.int32),
            pltpu.VMEM((CHUNK,), jnp.int32),
            pltpu.VMEM((CHUNK, 2 * HID), jnp.float32),
            pltpu.VMEM((CHUNK, 2 * HID), jnp.float32),
            pltpu.VMEM((CHUNK * 16,), jnp.float32),
            pltpu.VMEM((CHUNK,), jnp.float32),
            pltpu.VMEM((HID,), jnp.float32),
            pltpu.SemaphoreType.DMA,
            pltpu.SemaphoreType.DMA,
        ],
    )
    def k(t_hbm, src_hbm, dst_hbm, w2_hbm, out_hbm,
          si_v, di_v, ra_v, rb_v, p_v, o_v, w2_v, sem_a, sem_b):
        wid = lax.axis_index("s") * NC + lax.axis_index("c")
        base = wid * E_PER_W
        pltpu.sync_copy(w2_hbm, w2_v)
        w2_regs = [w2_v[pl.ds(16 * j, 16)] for j in range(HID // 16)]
        lanes16 = lax.iota(jnp.int32, 16)

        def chunk_body(g, _):
            off = base + g * CHUNK
            pltpu.sync_copy(src_hbm.at[pl.ds(off, CHUNK)], si_v)
            pltpu.sync_copy(dst_hbm.at[pl.ds(off, CHUNK)], di_v)
            ca = pltpu.async_copy(t_hbm.at[si_v], ra_v, sem_a)
            cb = pltpu.async_copy(t_hbm.at[di_v], rb_v, sem_b)
            ca.wait()
            cb.wait()

            def edge_body(e, _):
                acc = None
                for j in range(HID // 16):
                    z = ra_v[e, pl.ds(16 * j, 16)] + rb_v[e, pl.ds(HID + 16 * j, 16)]
                    t = jnp.maximum(z, 0.0) * w2_regs[j]
                    acc = t if acc is None else acc + t
                p_v[pl.ds(e * 16, 16)] = acc
                return 0

            lax.fori_loop(0, CHUNK, edge_body, 0)

            def grp_body(g2, _):
                flat0 = (g2 * 16 + lanes16) * 16
                tot = None
                for l in range(16):
                    v = plsc.load_gather(p_v, [flat0 + l])
                    tot = v if tot is None else tot + v
                o_v[pl.ds(g2 * 16, 16)] = tot
                return 0

            lax.fori_loop(0, CHUNK // 16, grp_body, 0)
            pltpu.sync_copy(o_v, out_hbm.at[pl.ds(off, CHUNK)])
            return 0

        lax.fori_loop(0, N_CHUNKS, chunk_body, 0)

    return k(T, src, dst, w2)


def kernel(node_feat, edge_candidate, num_edge_candidate, nnodes,
           W_enc, b_enc, W1, b1, W2, b2):
    T = _node_projections(node_feat, W_enc, b_enc, W1, b1)
    src = edge_candidate[:, 0]
    dst = edge_candidate[:, 1]
    scores = _edge_scores(T, src, dst, W2[:, 0])
    out = (scores + b2)[:, None]
    edge_rel = jnp.concatenate(
        [jnp.zeros((1,), dtype=nnodes.dtype), jnp.cumsum(nnodes)[:-1]])
    offsets = jnp.repeat(edge_rel, num_edge_candidate,
                         total_repeat_length=E_TOTAL)
    edge_candidate_idx = edge_candidate + offsets[:, None]
    return (out, edge_candidate_idx)


# speedup vs baseline: 6.3195x; 1.2629x over previous
import functools

import jax
import jax.numpy as jnp
from jax import lax
from jax.experimental import pallas as pl
from jax.experimental.pallas import tpu as pltpu
from jax.experimental.pallas import tpu_sc as plsc

N_NODES = 10000
E_TOTAL = 320000
D = 128
HID = 64

NC = 2
NS = 16
NW = NC * NS
E_PER_W = E_TOTAL // NW
CHUNK = 200
N_CHUNKS = E_PER_W // CHUNK
N_GRP = (CHUNK + 15) // 16
CHUNK_PAD = N_GRP * 16


def _node_projections(node_feat, W_enc, b_enc, W1, b1):
    def body(nf, we, be, w1c, bc, t_out):
        x = jnp.dot(nf[:], we[:], preferred_element_type=jnp.float32) + be[:]
        t_out[:] = jnp.dot(x, w1c[:], preferred_element_type=jnp.float32) + bc[:]

    w1cat = jnp.concatenate([W1[:D, :], W1[D:, :]], axis=1)
    bcat = jnp.concatenate([b1, jnp.zeros((HID,), jnp.float32)])
    return pl.pallas_call(
        body,
        out_shape=jax.ShapeDtypeStruct((N_NODES, 2 * HID), jnp.float32),
    )(node_feat, W_enc, b_enc.reshape(1, D), w1cat, bcat.reshape(1, 2 * HID))


def _edge_scores(T, src, dst, w2):
    mesh = plsc.VectorSubcoreMesh(
        core_axis_name="c", subcore_axis_name="s", num_cores=NC, num_subcores=NS
    )

    @functools.partial(
        pl.kernel,
        out_type=jax.ShapeDtypeStruct((E_TOTAL,), jnp.float32),
        mesh=mesh,
        compiler_params=pltpu.CompilerParams(needs_layout_passes=False),
        scratch_types=(
            [pltpu.VMEM((CHUNK,), jnp.int32)] * 4
            + [pltpu.VMEM((CHUNK, 2 * HID), jnp.float32)] * 4
            + [
                pltpu.VMEM((CHUNK_PAD * 16,), jnp.float32),
                pltpu.VMEM((CHUNK_PAD,), jnp.float32),
                pltpu.VMEM((HID,), jnp.float32),
            ]
            + [pltpu.SemaphoreType.DMA] * 4
        ),
    )
    def k(t_hbm, src_hbm, dst_hbm, w2_hbm, out_hbm,
          si0, di0, si1, di1, ra0, rb0, ra1, rb1,
          p_v, o_v, w2_v, sa0, sb0, sa1, sb1):
        wid = lax.axis_index("s") * NC + lax.axis_index("c")
        base = wid * E_PER_W
        pltpu.sync_copy(w2_hbm, w2_v)
        w2_regs = [w2_v[pl.ds(16 * j, 16)] for j in range(HID // 16)]
        lanes16 = lax.iota(jnp.int32, 16)
        bufs = [(si0, di0, ra0, rb0, sa0, sb0), (si1, di1, ra1, rb1, sa1, sb1)]

        def issue(c, b):
            si, di, ra, rb, sa, sb = bufs[b]
            off = base + c * CHUNK
            pltpu.sync_copy(src_hbm.at[pl.ds(off, CHUNK)], si)
            pltpu.sync_copy(dst_hbm.at[pl.ds(off, CHUNK)], di)
            pltpu.async_copy(t_hbm.at[si], ra, sa)
            pltpu.async_copy(t_hbm.at[di], rb, sb)

        def wait(b):
            si, di, ra, rb, sa, sb = bufs[b]
            pltpu.make_async_copy(t_hbm.at[si], ra, sa).wait()
            pltpu.make_async_copy(t_hbm.at[di], rb, sb).wait()

        def compute(c, b):
            _, _, ra_v, rb_v, _, _ = bufs[b]
            def edge_body(e, _):
                acc = None
                for j in range(HID // 16):
                    z = ra_v[e, pl.ds(16 * j, 16)] + rb_v[e, pl.ds(HID + 16 * j, 16)]
                    t = jnp.maximum(z, 0.0) * w2_regs[j]
                    acc = t if acc is None else acc + t
                p_v[pl.ds(e * 16, 16)] = acc
                return 0

            lax.fori_loop(0, CHUNK, edge_body, 0)

            def grp_body(g2, _):
                flat0 = (g2 * 16 + lanes16) * 16
                tot = None
                for l in range(16):
                    v = plsc.load_gather(p_v, [flat0 + l])
                    tot = v if tot is None else tot + v
                o_v[pl.ds(g2 * 16, 16)] = tot
                return 0

            lax.fori_loop(0, N_GRP, grp_body, 0)
            pltpu.sync_copy(o_v.at[pl.ds(0, CHUNK)],
                            out_hbm.at[pl.ds(base + c * CHUNK, CHUNK)])

        issue(0, 0)
        issue(1, 1)

        def loop_body(g2, _):
            for b in range(2):
                c = g2 * 2 + b
                wait(b)
                compute(c, b)

                @pl.when(c + 2 < N_CHUNKS)
                def _():
                    issue(c + 2, b)
            return 0

        lax.fori_loop(0, N_CHUNKS // 2, loop_body, 0)

    return k(T, src, dst, w2)


def kernel(node_feat, edge_candidate, num_edge_candidate, nnodes,
           W_enc, b_enc, W1, b1, W2, b2):
    T = _node_projections(node_feat, W_enc, b_enc, W1, b1)
    src = edge_candidate[:, 0]
    dst = edge_candidate[:, 1]
    scores = _edge_scores(T, src, dst, W2[:, 0])
    out = (scores + b2)[:, None]
    edge_rel = jnp.concatenate(
        [jnp.zeros((1,), dtype=nnodes.dtype), jnp.cumsum(nnodes)[:-1]])
    offsets = jnp.repeat(edge_rel, num_edge_candidate,
                         total_repeat_length=E_TOTAL)
    edge_candidate_idx = edge_candidate + offsets[:, None]
    return (out, edge_candidate_idx)


# speedup vs baseline: 7.7308x; 1.2233x over previous
import functools

import jax
import jax.numpy as jnp
from jax import lax
from jax.experimental import pallas as pl
from jax.experimental.pallas import tpu as pltpu
from jax.experimental.pallas import tpu_sc as plsc

N_NODES = 10000
E_TOTAL = 320000
D = 128
HID = 64
HID_W = HID // 2

NC = 2
NS = 16
NW = NC * NS
E_PER_W = E_TOTAL // NW
CHUNK = 200
N_CHUNKS = E_PER_W // CHUNK
N_GRP = (CHUNK + 15) // 16
CHUNK_PAD = N_GRP * 16


def _node_projections(node_feat, W_enc, b_enc, W1, b1):
    def body(nf, we, be, w1c, bc, t_out):
        x = jnp.dot(nf[:], we[:], preferred_element_type=jnp.float32) + be[:]
        t = jnp.dot(x, w1c[:], preferred_element_type=jnp.float32) + bc[:]
        t_out[:] = t.astype(jnp.bfloat16)

    w1cat = jnp.concatenate([W1[:D, :], W1[D:, :]], axis=1)
    bcat = jnp.concatenate([b1, jnp.zeros((HID,), jnp.float32)])
    return pl.pallas_call(
        body,
        out_shape=jax.ShapeDtypeStruct((N_NODES, 2 * HID), jnp.bfloat16),
    )(node_feat, W_enc, b_enc.reshape(1, D), w1cat, bcat.reshape(1, 2 * HID))


def _edge_scores(Tp, src, dst, w2p):
    mesh = plsc.VectorSubcoreMesh(
        core_axis_name="c", subcore_axis_name="s", num_cores=NC, num_subcores=NS
    )

    @functools.partial(
        pl.kernel,
        out_type=jax.ShapeDtypeStruct((E_TOTAL,), jnp.float32),
        mesh=mesh,
        compiler_params=pltpu.CompilerParams(
            needs_layout_passes=False, use_tc_tiling_on_sc=False
        ),
        scratch_types=(
            [
                pltpu.VMEM((E_PER_W,), jnp.int32),
                pltpu.VMEM((E_PER_W,), jnp.int32),
            ]
            + [pltpu.VMEM((CHUNK, 2 * HID_W), jnp.int32)] * 4
            + [
                pltpu.VMEM((CHUNK_PAD * 16,), jnp.float32),
                pltpu.VMEM((CHUNK_PAD,), jnp.float32),
                pltpu.VMEM((HID_W,), jnp.int32),
            ]
            + [pltpu.SemaphoreType.DMA] * 4
        ),
    )
    def k(t_hbm, src_hbm, dst_hbm, w2_hbm, out_hbm,
          si_v, di_v, ra0, rb0, ra1, rb1,
          p_v, o_v, w2_v, sa0, sb0, sa1, sb1):
        wid = lax.axis_index("s") * NC + lax.axis_index("c")
        base = wid * E_PER_W
        pltpu.sync_copy(w2_hbm, w2_v)
        pltpu.sync_copy(src_hbm.at[pl.ds(base, E_PER_W)], si_v)
        pltpu.sync_copy(dst_hbm.at[pl.ds(base, E_PER_W)], di_v)
        w2_regs = [
            plsc.bitcast(w2_v[pl.ds(16 * j, 16)], jnp.bfloat16)
            for j in range(HID_W // 16)
        ]
        lanes16 = lax.iota(jnp.int32, 16)
        bufs = [(ra0, rb0, sa0, sb0), (ra1, rb1, sa1, sb1)]

        def issue(c, b):
            ra, rb, sa, sb = bufs[b]
            off = c * CHUNK
            pltpu.async_copy(t_hbm.at[si_v.at[pl.ds(off, CHUNK)]], ra, sa)
            pltpu.async_copy(t_hbm.at[di_v.at[pl.ds(off, CHUNK)]], rb, sb)

        def wait(c, b):
            ra, rb, sa, sb = bufs[b]
            off = c * CHUNK
            pltpu.make_async_copy(t_hbm.at[si_v.at[pl.ds(off, CHUNK)]], ra, sa).wait()
            pltpu.make_async_copy(t_hbm.at[di_v.at[pl.ds(off, CHUNK)]], rb, sb).wait()

        def compute(c, b):
            ra_v, rb_v, _, _ = bufs[b]

            def edge_body(e, _):
                acc = None
                for j in range(HID_W // 16):
                    a = plsc.bitcast(ra_v[e, pl.ds(16 * j, 16)], jnp.bfloat16)
                    bb = plsc.bitcast(
                        rb_v[e, pl.ds(HID_W + 16 * j, 16)], jnp.bfloat16)
                    t = jnp.maximum(a + bb, jnp.bfloat16(0.0)) * w2_regs[j]
                    ta, tb = plsc.unpack(t, format=plsc.PackFormat.INTERLEAVED,
                                         preferred_element_type=jnp.float32)
                    part = ta + tb
                    acc = part if acc is None else acc + part
                p_v[pl.ds(e * 16, 16)] = acc
                return 0

            lax.fori_loop(0, CHUNK, edge_body, 0)

            def grp_body(g2, _):
                flat0 = (g2 * 16 + lanes16) * 16
                tot = None
                for l in range(16):
                    v = plsc.load_gather(p_v, [flat0 + l])
                    tot = v if tot is None else tot + v
                o_v[pl.ds(g2 * 16, 16)] = tot
                return 0

            lax.fori_loop(0, N_GRP, grp_body, 0)
            pltpu.sync_copy(o_v.at[pl.ds(0, CHUNK)],
                            out_hbm.at[pl.ds(base + c * CHUNK, CHUNK)])

        issue(0, 0)
        issue(1, 1)

        def loop_body(g2, _):
            for b in range(2):
                c = g2 * 2 + b
                wait(c, b)
                compute(c, b)

                @pl.when(c + 2 < N_CHUNKS)
                def _():
                    issue(c + 2, b)
            return 0

        lax.fori_loop(0, N_CHUNKS // 2, loop_body, 0)

    return k(Tp, src, dst, w2p)


def kernel(node_feat, edge_candidate, num_edge_candidate, nnodes,
           W_enc, b_enc, W1, b1, W2, b2):
    T = _node_projections(node_feat, W_enc, b_enc, W1, b1)
    Tp = jax.lax.bitcast_convert_type(
        T.reshape(N_NODES, HID, 2), jnp.int32)
    w2p = jax.lax.bitcast_convert_type(
        W2[:, 0].astype(jnp.bfloat16).reshape(HID_W, 2), jnp.int32)
    src = edge_candidate[:, 0]
    dst = edge_candidate[:, 1]
    scores = _edge_scores(Tp, src, dst, w2p)
    out = (scores + b2)[:, None]
    edge_rel = jnp.concatenate(
        [jnp.zeros((1,), dtype=nnodes.dtype), jnp.cumsum(nnodes)[:-1]])
    offsets = jnp.repeat(edge_rel, num_edge_candidate,
                         total_repeat_length=E_TOTAL)
    edge_candidate_idx = edge_candidate + offsets[:, None]
    return (out, edge_candidate_idx)


# speedup vs baseline: 10.2611x; 1.3273x over previous
import functools

import jax
import jax.numpy as jnp
from jax import lax
from jax.experimental import pallas as pl
from jax.experimental.pallas import tpu as pltpu
from jax.experimental.pallas import tpu_sc as plsc

N_NODES = 10000
E_TOTAL = 320000
D = 128
HID = 64
HID_W = HID // 2

NC = 2
NS = 16
NW = NC * NS
E_PER_W = E_TOTAL // NW
CHUNK = 200
N_CHUNKS = E_PER_W // CHUNK
N_GRP = (CHUNK + 15) // 16
CHUNK_PAD = N_GRP * 16


def _node_projections(node_feat, W_enc, b_enc, W1, b1):
    def body(nf, we, be, w1c, bc, t_out):
        x = jnp.dot(nf[:], we[:], preferred_element_type=jnp.float32) + be[:]
        t = jnp.dot(x, w1c[:], preferred_element_type=jnp.float32) + bc[:]
        t_out[:] = t.astype(jnp.bfloat16)

    w1cat = jnp.concatenate([W1[:D, :], W1[D:, :]], axis=1)
    bcat = jnp.concatenate([b1, jnp.zeros((HID,), jnp.float32)])
    return pl.pallas_call(
        body,
        out_shape=jax.ShapeDtypeStruct((N_NODES, 2 * HID), jnp.bfloat16),
    )(node_feat, W_enc, b_enc.reshape(1, D), w1cat, bcat.reshape(1, 2 * HID))


def _edge_scores(Tp, src, dst, w2p):
    mesh = plsc.VectorSubcoreMesh(
        core_axis_name="c", subcore_axis_name="s", num_cores=NC, num_subcores=NS
    )

    @functools.partial(
        pl.kernel,
        out_type=jax.ShapeDtypeStruct((E_TOTAL,), jnp.float32),
        mesh=mesh,
        compiler_params=pltpu.CompilerParams(
            needs_layout_passes=False, use_tc_tiling_on_sc=False
        ),
        scratch_types=(
            [
                pltpu.VMEM((E_PER_W,), jnp.int32),
                pltpu.VMEM((E_PER_W,), jnp.int32),
            ]
            + [pltpu.VMEM((CHUNK, 2 * HID_W), jnp.int32)] * 4
            + [
                pltpu.VMEM((CHUNK_PAD * 16,), jnp.float32),
                pltpu.VMEM((CHUNK_PAD,), jnp.float32),
                pltpu.VMEM((HID_W,), jnp.int32),
            ]
            + [pltpu.SemaphoreType.DMA] * 4
        ),
    )
    def k(t_hbm, src_hbm, dst_hbm, w2_hbm, out_hbm,
          si_v, di_v, ra0, rb0, ra1, rb1,
          p_v, o_v, w2_v, sa0, sb0, sa1, sb1):
        wid = lax.axis_index("s") * NC + lax.axis_index("c")
        base = wid * E_PER_W
        pltpu.sync_copy(w2_hbm, w2_v)
        pltpu.sync_copy(src_hbm.at[pl.ds(base, E_PER_W)], si_v)
        pltpu.sync_copy(dst_hbm.at[pl.ds(base, E_PER_W)], di_v)
        w2_regs = [
            plsc.bitcast(w2_v[pl.ds(16 * j, 16)], jnp.bfloat16)
            for j in range(HID_W // 16)
        ]
        lanes16 = lax.iota(jnp.int32, 16)
        bufs = [(ra0, rb0, sa0, sb0), (ra1, rb1, sa1, sb1)]

        def issue(c, b):
            ra, rb, sa, sb = bufs[b]
            off = c * CHUNK
            pltpu.async_copy(t_hbm.at[si_v.at[pl.ds(off, CHUNK)]], ra, sa)
            pltpu.async_copy(t_hbm.at[di_v.at[pl.ds(off, CHUNK)]], rb, sb)

        def wait(c, b):
            ra, rb, sa, sb = bufs[b]
            off = c * CHUNK
            pltpu.make_async_copy(t_hbm.at[si_v.at[pl.ds(off, CHUNK)]], ra, sa).wait()
            pltpu.make_async_copy(t_hbm.at[di_v.at[pl.ds(off, CHUNK)]], rb, sb).wait()

        def compute(c, b):
            ra_v, rb_v, _, _ = bufs[b]

            G = 4
            NJ = HID_W // 16

            def edge_body(i, _):
                e0 = i * G
                a_regs = [
                    [plsc.bitcast(ra_v[e0 + g, pl.ds(16 * j, 16)], jnp.bfloat16)
                     for j in range(NJ)]
                    for g in range(G)
                ]
                b_regs = [
                    [plsc.bitcast(rb_v[e0 + g, pl.ds(HID_W + 16 * j, 16)],
                                  jnp.bfloat16)
                     for j in range(NJ)]
                    for g in range(G)
                ]
                for g in range(G):
                    acc = None
                    for j in range(NJ):
                        t = jnp.maximum(a_regs[g][j] + b_regs[g][j],
                                        jnp.bfloat16(0.0)) * w2_regs[j]
                        ta, tb = plsc.unpack(
                            t, format=plsc.PackFormat.INTERLEAVED,
                            preferred_element_type=jnp.float32)
                        part = ta + tb
                        acc = part if acc is None else acc + part
                    p_v[pl.ds((e0 + g) * 16, 16)] = acc
                return 0

            lax.fori_loop(0, CHUNK // G, edge_body, 0)

            def grp_body(g2, _):
                flat0 = (g2 * 16 + lanes16) * 16
                tot = None
                for l in range(16):
                    v = plsc.load_gather(p_v, [flat0 + l])
                    tot = v if tot is None else tot + v
                o_v[pl.ds(g2 * 16, 16)] = tot
                return 0

            lax.fori_loop(0, N_GRP, grp_body, 0)
            pltpu.sync_copy(o_v.at[pl.ds(0, CHUNK)],
                            out_hbm.at[pl.ds(base + c * CHUNK, CHUNK)])

        issue(0, 0)
        issue(1, 1)

        def loop_body(g2, _):
            for b in range(2):
                c = g2 * 2 + b
                wait(c, b)
                compute(c, b)

                @pl.when(c + 2 < N_CHUNKS)
                def _():
                    issue(c + 2, b)
            return 0

        lax.fori_loop(0, N_CHUNKS // 2, loop_body, 0)

    return k(Tp, src, dst, w2p)


def kernel(node_feat, edge_candidate, num_edge_candidate, nnodes,
           W_enc, b_enc, W1, b1, W2, b2):
    T = _node_projections(node_feat, W_enc, b_enc, W1, b1)
    Tp = jax.lax.bitcast_convert_type(
        T.reshape(N_NODES, HID, 2), jnp.int32)
    w2p = jax.lax.bitcast_convert_type(
        W2[:, 0].astype(jnp.bfloat16).reshape(HID_W, 2), jnp.int32)
    src = edge_candidate[:, 0]
    dst = edge_candidate[:, 1]
    scores = _edge_scores(Tp, src, dst, w2p)
    out = (scores + b2)[:, None]
    edge_rel = jnp.concatenate(
        [jnp.zeros((1,), dtype=nnodes.dtype), jnp.cumsum(nnodes)[:-1]])
    offsets = jnp.repeat(edge_rel, num_edge_candidate,
                         total_repeat_length=E_TOTAL)
    edge_candidate_idx = edge_candidate + offsets[:, None]
    return (out, edge_candidate_idx)


# speedup vs baseline: 10.5900x; 1.0320x over previous
import functools

import jax
import jax.numpy as jnp
from jax import lax
from jax.experimental import pallas as pl
from jax.experimental.pallas import tpu as pltpu
from jax.experimental.pallas import tpu_sc as plsc

N_NODES = 10000
E_TOTAL = 320000
D = 128
HID = 64
HID_W = HID // 2

NC = 2
NS = 16
NW = NC * NS
E_PER_W = E_TOTAL // NW
CHUNK = 200
N_CHUNKS = E_PER_W // CHUNK
N_GRP = (CHUNK + 15) // 16
CHUNK_PAD = N_GRP * 16


def _node_projections(node_feat, W_enc, b_enc, W1, b1):
    def body(nf, we, be, w1c, bc, t_out):
        x = jnp.dot(nf[:], we[:], preferred_element_type=jnp.float32) + be[:]
        t = jnp.dot(x, w1c[:], preferred_element_type=jnp.float32) + bc[:]
        t_out[:] = t.astype(jnp.bfloat16)

    w1cat = jnp.concatenate([W1[:D, :], W1[D:, :]], axis=1)
    bcat = jnp.concatenate([b1, jnp.zeros((HID,), jnp.float32)])
    return pl.pallas_call(
        body,
        out_shape=jax.ShapeDtypeStruct((N_NODES, 2 * HID), jnp.bfloat16),
    )(node_feat, W_enc, b_enc.reshape(1, D), w1cat, bcat.reshape(1, 2 * HID))


def _edge_scores(Tp, src, dst, w2p):
    mesh = plsc.VectorSubcoreMesh(
        core_axis_name="c", subcore_axis_name="s", num_cores=NC, num_subcores=NS
    )

    @functools.partial(
        pl.kernel,
        out_type=jax.ShapeDtypeStruct((E_TOTAL,), jnp.float32),
        mesh=mesh,
        compiler_params=pltpu.CompilerParams(
            needs_layout_passes=False, use_tc_tiling_on_sc=False
        ),
        scratch_types=(
            [
                pltpu.VMEM((E_PER_W,), jnp.int32),
                pltpu.VMEM((E_PER_W,), jnp.int32),
            ]
            + [pltpu.VMEM((CHUNK, 2 * HID_W), jnp.int32)] * 8
            + [
                pltpu.VMEM((CHUNK_PAD * 16,), jnp.float32),
                pltpu.VMEM((HID_W,), jnp.int32),
            ]
            + [pltpu.VMEM((CHUNK_PAD,), jnp.float32)] * 4
            + [pltpu.SemaphoreType.DMA] * 12
        ),
    )
    def k(t_hbm, src_hbm, dst_hbm, w2_hbm, out_hbm,
          si_v, di_v, ra0, rb0, ra1, rb1, ra2, rb2, ra3, rb3,
          p_v, w2_v, o0, o1, o2, o3,
          sa0, sb0, sa1, sb1, sa2, sb2, sa3, sb3, so0, so1, so2, so3):
        wid = lax.axis_index("s") * NC + lax.axis_index("c")
        base = wid * E_PER_W
        pltpu.sync_copy(w2_hbm, w2_v)
        pltpu.sync_copy(src_hbm.at[pl.ds(base, E_PER_W)], si_v)
        pltpu.sync_copy(dst_hbm.at[pl.ds(base, E_PER_W)], di_v)
        w2_regs = [
            plsc.bitcast(w2_v[pl.ds(16 * j, 16)], jnp.bfloat16)
            for j in range(HID_W // 16)
        ]
        lanes16 = lax.iota(jnp.int32, 16)
        bufs = [
            (ra0, rb0, sa0, sb0, o0, so0),
            (ra1, rb1, sa1, sb1, o1, so1),
            (ra2, rb2, sa2, sb2, o2, so2),
            (ra3, rb3, sa3, sb3, o3, so3),
        ]
        RING = len(bufs)

        def issue(c, b):
            ra, rb, sa, sb, _, _ = bufs[b]
            off = c * CHUNK
            pltpu.async_copy(t_hbm.at[si_v.at[pl.ds(off, CHUNK)]], ra, sa)
            pltpu.async_copy(t_hbm.at[di_v.at[pl.ds(off, CHUNK)]], rb, sb)

        def wait(c, b):
            ra, rb, sa, sb, _, _ = bufs[b]
            off = c * CHUNK
            pltpu.make_async_copy(t_hbm.at[si_v.at[pl.ds(off, CHUNK)]], ra, sa).wait()
            pltpu.make_async_copy(t_hbm.at[di_v.at[pl.ds(off, CHUNK)]], rb, sb).wait()

        def wait_out(c, b):
            _, _, _, _, o_v, so = bufs[b]
            pltpu.make_async_copy(
                o_v.at[pl.ds(0, CHUNK)],
                out_hbm.at[pl.ds(base + c * CHUNK, CHUNK)], so).wait()

        def compute(c, b):
            ra_v, rb_v, _, _, o_v, so = bufs[b]

            G = 4
            NJ = HID_W // 16

            def edge_body(i, _):
                e0 = i * G
                a_regs = [
                    [plsc.bitcast(ra_v[e0 + g, pl.ds(16 * j, 16)], jnp.bfloat16)
                     for j in range(NJ)]
                    for g in range(G)
                ]
                b_regs = [
                    [plsc.bitcast(rb_v[e0 + g, pl.ds(HID_W + 16 * j, 16)],
                                  jnp.bfloat16)
                     for j in range(NJ)]
                    for g in range(G)
                ]
                for g in range(G):
                    ts = None
                    for j in range(NJ):
                        t = jnp.maximum(a_regs[g][j] + b_regs[g][j],
                                        jnp.bfloat16(0.0)) * w2_regs[j]
                        ts = t if ts is None else ts + t
                    ta, tb = plsc.unpack(
                        ts, format=plsc.PackFormat.INTERLEAVED,
                        preferred_element_type=jnp.float32)
                    p_v[pl.ds((e0 + g) * 16, 16)] = ta + tb
                return 0

            lax.fori_loop(0, CHUNK // G, edge_body, 0)

            def grp_body(g2, _):
                flat0 = (g2 * 16 + lanes16) * 16
                tot = None
                for l in range(16):
                    v = plsc.load_gather(p_v, [flat0 + l])
                    tot = v if tot is None else tot + v
                o_v[pl.ds(g2 * 16, 16)] = tot
                return 0

            lax.fori_loop(0, N_GRP, grp_body, 0)
            pltpu.async_copy(o_v.at[pl.ds(0, CHUNK)],
                             out_hbm.at[pl.ds(base + c * CHUNK, CHUNK)], so)

        for b in range(RING):
            issue(b, b)

        N_MAIN = (N_CHUNKS // RING) * RING

        def loop_body(q, _):
            for b in range(RING):
                c = q * RING + b
                wait(c, b)

                @pl.when(c >= RING)
                def _():
                    wait_out(c - RING, b)

                compute(c, b)

                @pl.when(c + RING < N_CHUNKS)
                def _():
                    issue(c + RING, b)
            return 0

        lax.fori_loop(0, N_MAIN // RING, loop_body, 0)
        for b in range(N_CHUNKS - N_MAIN):
            c = N_MAIN + b
            wait(c, b)
            wait_out(c - RING, b)
            compute(c, b)
        for b in range(N_CHUNKS - N_MAIN, RING):
            wait_out(N_MAIN - RING + b, b)
        for b in range(N_CHUNKS - N_MAIN):
            wait_out(N_MAIN + b, b)

    return k(Tp, src, dst, w2p)


def kernel(node_feat, edge_candidate, num_edge_candidate, nnodes,
           W_enc, b_enc, W1, b1, W2, b2):
    T = _node_projections(node_feat, W_enc, b_enc, W1, b1)
    Tp = jax.lax.bitcast_convert_type(
        T.reshape(N_NODES, HID, 2), jnp.int32)
    w2p = jax.lax.bitcast_convert_type(
        W2[:, 0].astype(jnp.bfloat16).reshape(HID_W, 2), jnp.int32)
    src = edge_candidate[:, 0]
    dst = edge_candidate[:, 1]
    scores = _edge_scores(Tp, src, dst, w2p)
    out = (scores + b2)[:, None]
    edge_rel = jnp.concatenate(
        [jnp.zeros((1,), dtype=nnodes.dtype), jnp.cumsum(nnodes)[:-1]])
    offsets = jnp.repeat(edge_rel, num_edge_candidate,
                         total_repeat_length=E_TOTAL)
    edge_candidate_idx = edge_candidate + offsets[:, None]
    return (out, edge_candidate_idx)
